# baseline (speedup 1.0000x reference)
import functools

import jax
import jax.numpy as jnp
from jax import lax
from jax.experimental import pallas as pl
from jax.experimental.pallas import tpu as pltpu
from jax.experimental.pallas import tpu_sc as plsc

B = 16384
K = 32
NROWS = 1000000
NPAD = 1000064

_NC = 2
_NS = 16
_NW = _NC * _NS
_RPW = NROWS // _NW
_CH = 256
_NCHUNK = 124
_NBIN = 16
_CPB = 8
_BCAP = 96
_CAP = 640
_CCAP = 160
_OW = _NW * _CAP
_DUMP = B
_SL = 4096
_VRING = 4


@functools.cache
def _make_gather_u():
    mesh = plsc.VectorSubcoreMesh(core_axis_name="c", subcore_axis_name="s")

    @functools.partial(
        pl.kernel,
        mesh=mesh,
        compiler_params=pltpu.CompilerParams(
            use_tc_tiling_on_sc=True, needs_layout_passes=False),
        out_type=[
            jax.ShapeDtypeStruct((K, _OW), jnp.float32),
            jax.ShapeDtypeStruct((_NW, 8, 128), jnp.int32),
        ],
        scratch_types=(
            [pltpu.VMEM((K, _CH), jnp.float32)] * 2
            + [pltpu.VMEM((_SL,), jnp.int32)]
            + [pltpu.VMEM((_CAP,), jnp.int32)] * 2
            + [pltpu.VMEM((_NBIN * _BCAP,), jnp.int32)] * 2
            + [pltpu.VMEM((_NBIN,), jnp.int32)]
            + [pltpu.VMEM((_CCAP,), jnp.int32)] * 2
            + [pltpu.VMEM((_CAP, K), jnp.float32)]
            + [pltpu.VMEM((8, 128), jnp.int32)]
            + [pltpu.SemaphoreType.DMA] * 2
        ),
    )
    def _gather_u(wt_hbm, ui_hbm, ut_out, pu_out,
                  cbu0, cbu1, ibuf, lru, lpu, bru, bpu, bcu, cru, cpu,
                  rau, p2u, semu0, semu1):
        c = lax.axis_index("c")
        s = lax.axis_index("s")
        wid = c * _NS + s
        lo = wid * _RPW
        hi = lo + _RPW
        ab = pl.multiple_of(lo - lax.rem(lo, 128), 128)
        lane = jax.lax.iota(jnp.int32, 16)
        zero16 = jnp.full((16,), 0, jnp.int32)

        cbu = (cbu0, cbu1)
        semu = (semu0, semu1)

        for m in range(64):
            row = zero16 + (m // 8)
            colv = lane + (m % 8) * 16
            plsc.store_scatter(p2u, [row, colv], lane + _DUMP)

        def _outer(n0, sl):
            pltpu.sync_copy(ui_hbm.at[pl.ds(sl * _SL, _SL)], ibuf)

            def _bin(g, n):
                ivec = ibuf[pl.ds(g * 16, 16)]
                pvec = lane + (sl * _SL + g * 16)
                mk = (ivec >= lo) & (ivec < hi)
                plsc.store_compressed(lru.at[pl.ds(n, 16)], ivec, mask=mk)
                plsc.store_compressed(lpu.at[pl.ds(n, 16)], pvec, mask=mk)
                cnt = plsc.all_reduce_population_count(mk)[0]
                return jnp.minimum(n + cnt, _CAP - 16)

            return lax.fori_loop(0, _SL // 16, _bin, n0)

        nu = jnp.int32(0)
        for sl in range(B // _SL):
            nu = _outer(nu, sl)
        nvu = (nu + 15) // 16

        for b in range(_NBIN):
            blo = ab + b * (_CPB * _CH)
            bhi = blo + (_CPB * _CH)

            def _bb(m, cnt, blo=blo, bhi=bhi, b=b):
                rvec = lru[pl.ds(m * 16, 16)]
                pvec = lpu[pl.ds(m * 16, 16)]
                valid = (lane + m * 16) < nu
                mk = valid & (rvec >= blo) & (rvec < bhi)
                off = b * _BCAP + cnt
                plsc.store_compressed(bru.at[pl.ds(off, 16)], rvec, mask=mk)
                plsc.store_compressed(bpu.at[pl.ds(off, 16)], pvec, mask=mk)
                return jnp.minimum(
                    cnt + plsc.all_reduce_population_count(mk)[0],
                    _BCAP - 16)

            cnt = lax.fori_loop(0, nvu, _bb, jnp.int32(0))
            plsc.store_scatter(bcu, [zero16 + b], zero16 + cnt,
                               mask=lane < 1)

        def _cstart(j, slot):
            cs = jnp.minimum(ab + j * _CH, NPAD - _CH)
            cs = pl.multiple_of(cs, 128)
            pltpu.make_async_copy(
                wt_hbm.at[:, pl.ds(cs, _CH)], cbu[slot], semu[slot]).start()

        def _cwait(slot):
            pltpu.make_async_copy(
                wt_hbm.at[:, pl.ds(0, _CH)], cbu[slot], semu[slot]).wait()

        def _extract(j, slot, qu):
            cs = jnp.minimum(ab + j * _CH, NPAD - _CH)
            bin_ = j // _CPB
            boff = bin_ * _BCAP
            bn = plsc.load_gather(bcu, [zero16 + bin_])[0]
            bvecs = (bn + 15) // 16

            def _mb(m, mc):
                rvec = bru[pl.ds(boff + m * 16, 16)]
                pvec = bpu[pl.ds(boff + m * 16, 16)]
                valid = (lane + m * 16) < bn
                mk = valid & (rvec >= cs) & (rvec < cs + _CH) \
                    & (rvec >= lo) & (rvec < hi)
                plsc.store_compressed(cru.at[pl.ds(mc, 16)], rvec, mask=mk)
                plsc.store_compressed(cpu.at[pl.ds(mc, 16)], pvec, mask=mk)
                return jnp.minimum(
                    mc + plsc.all_reduce_population_count(mk)[0],
                    _CCAP - 32)

            mcu = lax.fori_loop(0, bvecs, _mb, jnp.int32(0))

            def _ex(t, q):
                rr = cru[pl.ds(t, 16)][0]
                pp = cpu[pl.ds(t, 16)][0]
                qt = jnp.minimum(q + t, _CAP - 1)
                col = zero16 + (rr - cs)
                dst = zero16 + qt
                for h in range(2):
                    rows = lane + h * 16
                    vec = plsc.load_gather(cbu[slot], [rows, col])
                    plsc.store_scatter(rau, [dst, rows], vec)
                plsc.store_scatter(p2u, [zero16 + (qt >> 7),
                                         zero16 + (qt & 127)],
                                   zero16 + pp, mask=lane < 1)
                return q

            qu = lax.fori_loop(0, mcu, _ex, qu) + mcu
            return jnp.minimum(qu, _CAP - 1)

        _cstart(0, 0)

        def _scanloop(j2, qu):
            _cstart(2 * j2 + 1, 1)
            _cwait(0)
            qu = _extract(2 * j2, 0, qu)
            _cstart(2 * j2 + 2, 0)
            _cwait(1)
            qu = _extract(2 * j2 + 1, 1, qu)
            return qu

        qu = lax.fori_loop(0, _NCHUNK // 2, _scanloop, jnp.int32(0))
        _cwait(0)

        plsc.store_scatter(p2u, [zero16 + 7, zero16 + 127],
                           zero16 + qu, mask=lane < 1)
        pltpu.sync_copy(p2u, pu_out.at[wid])

        obase = wid * _CAP

        def _tpose(g, _):
            for jj in range(16):
                src = zero16 + (g * 16 + jj)
                colv = zero16 + lax.rem(g * 16 + jj, 128)
                for h in range(2):
                    rows = lane + h * 16
                    vec = plsc.load_gather(rau, [src, rows])
                    plsc.store_scatter(cbu0, [rows, colv], vec)
            return _

        for piece in range(_CAP // 128):
            lax.fori_loop(piece * 8, (piece + 1) * 8, _tpose, None)
            pltpu.sync_copy(
                cbu0.at[:, pl.ds(0, 128)],
                ut_out.at[:, pl.ds(obase + piece * 128, 128)])

    return _gather_u


@functools.cache
def _make_gather_v():
    mesh = plsc.VectorSubcoreMesh(core_axis_name="c", subcore_axis_name="s")

    @functools.partial(
        pl.kernel,
        mesh=mesh,
        compiler_params=pltpu.CompilerParams(
            use_tc_tiling_on_sc=True, needs_layout_passes=False),
        out_type=jax.ShapeDtypeStruct((K, _OW), jnp.float32),
        scratch_types=(
            [pltpu.VMEM((K, 128), jnp.float32)] * _VRING
            + [pltpu.VMEM((B + 128,), jnp.int32)]
            + [pltpu.VMEM((_CAP, K), jnp.float32)]
            + [pltpu.VMEM((K, _CH), jnp.float32)]
            + [pltpu.VMEM((8, 128), jnp.int32)]
            + [pltpu.SemaphoreType.DMA] * _VRING
        ),
    )
    def _gather_v(ht_hbm, vi_hbm, pu_hbm, vt_out,
                  vb0, vb1, vb2, vb3, viv, rav, tbuf, p2,
                  sv0, sv1, sv2, sv3):
        c = lax.axis_index("c")
        s = lax.axis_index("s")
        wid = c * _NS + s
        lane = jax.lax.iota(jnp.int32, 16)
        zero16 = jnp.full((16,), 0, jnp.int32)
        vblk = (vb0, vb1, vb2, vb3)
        vsem = (sv0, sv1, sv2, sv3)

        pltpu.sync_copy(pu_hbm.at[wid], p2)
        for sl in range(B // _SL):
            pltpu.sync_copy(vi_hbm.at[pl.ds(sl * _SL, _SL)],
                            viv.at[pl.ds(sl * _SL, _SL)])
        qu = plsc.load_gather(p2, [zero16 + 7, zero16 + 127])[0]

        def _vidx(t):
            pr = zero16 + (t >> 7)
            pc = zero16 + (t & 127)
            pos = plsc.load_gather(p2, [pr, pc])[0]
            vv = viv[pl.ds(pos, 16)][0]
            return jnp.minimum(jnp.maximum(vv, 0), NROWS - 1)

        def _vstart(t, j):
            vv = _vidx(t)
            c0 = pl.multiple_of(vv - lax.rem(vv, 128), 128)
            slot = j % _VRING
            pltpu.make_async_copy(
                ht_hbm.at[:, pl.ds(c0, 128)], vblk[slot], vsem[slot]).start()

        def _vfinish(t, j):
            slot = j % _VRING
            pltpu.make_async_copy(
                ht_hbm.at[:, pl.ds(0, 128)], vblk[slot], vsem[slot]).wait()
            vv = _vidx(t)
            col = zero16 + lax.rem(vv, 128)
            dst = zero16 + t
            for h in range(2):
                rows = lane + h * 16
                vec = plsc.load_gather(vblk[slot], [rows, col])
                plsc.store_scatter(rav, [dst, rows], vec)

        ngrp = (qu + 15) // 16

        def _vbody(g, _):
            for j in range(16):
                t = g * 16 + j
                if j < _VRING:
                    @pl.when(g >= 1)
                    def _fin(t=t, j=j):
                        _vfinish(t - _VRING, j + 16 - _VRING)
                else:
                    _vfinish(t - _VRING, j - _VRING)
                _vstart(t, j)
            return _

        lax.fori_loop(0, ngrp, _vbody, None)
        for j in range(_VRING):
            @pl.when(ngrp >= 1)
            def _tail(j=j):
                _vfinish(ngrp * 16 - _VRING + j, j + 16 - _VRING)

        obase = wid * _CAP

        def _tpose(g, _):
            for jj in range(16):
                src = zero16 + (g * 16 + jj)
                colv = zero16 + lax.rem(g * 16 + jj, 128)
                for h in range(2):
                    rows = lane + h * 16
                    vec = plsc.load_gather(rav, [src, rows])
                    plsc.store_scatter(tbuf, [rows, colv], vec)
            return _

        for piece in range(_CAP // 128):
            lax.fori_loop(piece * 8, (piece + 1) * 8, _tpose, None)
            pltpu.sync_copy(
                tbuf.at[:, pl.ds(0, 128)],
                vt_out.at[:, pl.ds(obase + piece * 128, 128)])

    return _gather_v


@functools.cache
def _make_scatter_sc():
    mesh = plsc.VectorSubcoreMesh(core_axis_name="c", subcore_axis_name="s")

    @functools.partial(
        pl.kernel,
        mesh=mesh,
        compiler_params=pltpu.CompilerParams(
            use_tc_tiling_on_sc=False, needs_layout_passes=False),
        out_type=jax.ShapeDtypeStruct((B + 128,), jnp.float32),
        scratch_types=[
            pltpu.VMEM((8, 128), jnp.int32),
            pltpu.VMEM((8, 128), jnp.float32),
            pltpu.SemaphoreType.DMA,
        ],
    )
    def _scatter_sc(logit_hbm, pos_hbm, out_hbm, p2, lbuf, sem):
        c = lax.axis_index("c")
        s = lax.axis_index("s")
        wid = c * _NS + s
        pltpu.sync_copy(pos_hbm.at[wid], p2)
        base = wid * _CAP
        copies = []
        for k in range(_CAP // 128):
            pltpu.sync_copy(
                logit_hbm.at[pl.ds(base + k * 128, 128)], lbuf.at[k])
            copies.append(pltpu.async_copy(
                lbuf.at[k], out_hbm.at[p2.at[k]], sem))
        for cp_ in copies:
            cp_.wait()

    return _scatter_sc


def _mlp_body(ut_ref, vt_ref, w1a_ref, w1b_ref, w2_ref, b2_ref, o_ref):
    ht = jnp.dot(w1a_ref[...], ut_ref[...], preferred_element_type=jnp.float32)
    ht = ht + jnp.dot(w1b_ref[...], vt_ref[...],
                      preferred_element_type=jnp.float32)
    ht = jnp.maximum(ht, 0.0)
    logit = jnp.dot(w2_ref[...], ht, preferred_element_type=jnp.float32)
    o_ref[...] = jax.nn.sigmoid(logit + b2_ref[...])


def _mlp_tc(ut, vt, w1a, w1b, w2, b2):
    return pl.pallas_call(
        _mlp_body,
        out_shape=jax.ShapeDtypeStruct((1, _OW), jnp.float32),
    )(ut, vt, w1a, w1b, w2, b2)


def kernel(x, W, H, W1, W2, b2):
    ui = x[:, 0].astype(jnp.int32)
    vi = x[:, 1].astype(jnp.int32)
    wt = jnp.transpose(W)
    ht = jnp.transpose(H)
    ut, pu = _make_gather_u()(wt, ui)
    vt = _make_gather_v()(ht, vi, pu)
    w1a = W1[:, :K]
    w1b = W1[:, K:]
    logit = _mlp_tc(ut, vt, w1a, w1b, W2, b2.reshape(1, 1))
    out = _make_scatter_sc()(logit.reshape(_OW), pu)
    return out[:B]


# speedup vs baseline: 4.1228x; 4.1228x over previous
import functools

import jax
import jax.numpy as jnp
from jax import lax
from jax.experimental import pallas as pl
from jax.experimental.pallas import tpu as pltpu
from jax.experimental.pallas import tpu_sc as plsc

B = 16384
K = 32
NROWS = 1000000

_NC = 2
_NS = 16
_NW = _NC * _NS
_BPW = B // _NW
_RING = 8


@functools.cache
def _make_gather_sc():
    mesh = plsc.VectorSubcoreMesh(core_axis_name="c", subcore_axis_name="s")

    @functools.partial(
        pl.kernel,
        mesh=mesh,
        compiler_params=pltpu.CompilerParams(
            use_tc_tiling_on_sc=True, needs_layout_passes=False),
        out_type=[
            jax.ShapeDtypeStruct((K, B), jnp.float32),
            jax.ShapeDtypeStruct((K, B), jnp.float32),
        ],
        scratch_types=(
            [pltpu.VMEM((_BPW,), jnp.int32),
             pltpu.VMEM((_BPW,), jnp.int32),
             pltpu.VMEM((K, _BPW), jnp.float32),
             pltpu.VMEM((K, _BPW), jnp.float32)]
            + [pltpu.VMEM((K, 128), jnp.float32)] * (2 * _RING)
            + [pltpu.SemaphoreType.DMA] * (2 * _RING)
        ),
    )
    def _gather_sc(wt_hbm, ht_hbm, ui_hbm, vi_hbm, ut_out, vt_out,
                   ui_v, vi_v, u_fm, v_fm, *ring):
        ublk = ring[0:_RING]
        vblk = ring[_RING:2 * _RING]
        usem = ring[2 * _RING:3 * _RING]
        vsem = ring[3 * _RING:4 * _RING]

        wid = lax.axis_index("s") * _NC + lax.axis_index("c")
        base = wid * _BPW
        pltpu.sync_copy(ui_hbm.at[pl.ds(base, _BPW)], ui_v)
        pltpu.sync_copy(vi_hbm.at[pl.ds(base, _BPW)], vi_v)

        lane = jax.lax.iota(jnp.int32, 16)
        rows_lo = lane
        rows_hi = lane + 16

        def _start(j, r, s):
            slot = j % _RING
            c0u = pl.multiple_of(r - (r % 128), 128)
            c0v = pl.multiple_of(s - (s % 128), 128)
            pltpu.make_async_copy(
                wt_hbm.at[:, pl.ds(c0u, 128)], ublk[slot], usem[slot]
            ).start()
            pltpu.make_async_copy(
                ht_hbm.at[:, pl.ds(c0v, 128)], vblk[slot], vsem[slot]
            ).start()

        def _finish(j, r, s, dstcol):
            slot = j % _RING
            pltpu.make_async_copy(
                wt_hbm.at[:, pl.ds(0, 128)], ublk[slot], usem[slot]
            ).wait()
            pltpu.make_async_copy(
                ht_hbm.at[:, pl.ds(0, 128)], vblk[slot], vsem[slot]
            ).wait()
            cu = jnp.full((16,), 0, jnp.int32) + (r % 128)
            cv = jnp.full((16,), 0, jnp.int32) + (s % 128)
            dc = jnp.full((16,), 0, jnp.int32) + dstcol
            for rows in (rows_lo, rows_hi):
                uvecf = plsc.load_gather(ublk[slot], [rows, cu])
                vvecf = plsc.load_gather(vblk[slot], [rows, cv])
                plsc.store_scatter(u_fm, [rows, dc], uvecf)
                plsc.store_scatter(v_fm, [rows, dc], vvecf)

        def _body(g, carry):
            upv, vpv = carry
            uvec = ui_v[pl.ds(g * 16, 16)]
            vvec = vi_v[pl.ds(g * 16, 16)]
            for j in range(16):
                if j < _RING:
                    @pl.when(g >= 1)
                    def _fin():
                        _finish(j, upv[j + 16 - _RING], vpv[j + 16 - _RING],
                                g * 16 + j - _RING)
                else:
                    _finish(j, uvec[j - _RING], vvec[j - _RING],
                            g * 16 + j - _RING)
                _start(j, uvec[j], vvec[j])
            return (uvec, vvec)

        zero16 = jnp.zeros((16,), jnp.int32)
        upv, vpv = lax.fori_loop(0, _BPW // 16, _body, (zero16, zero16))

        for j in range(_RING):
            _finish(j, upv[j + 16 - _RING], vpv[j + 16 - _RING],
                    _BPW + j - _RING)

        pltpu.sync_copy(u_fm, ut_out.at[:, pl.ds(base, _BPW)])
        pltpu.sync_copy(v_fm, vt_out.at[:, pl.ds(base, _BPW)])

    return _gather_sc


def _mlp_body(ut_ref, vt_ref, w1a_ref, w1b_ref, w2_ref, b2_ref, o_ref):
    ht = jnp.dot(w1a_ref[...], ut_ref[...], preferred_element_type=jnp.float32)
    ht = ht + jnp.dot(w1b_ref[...], vt_ref[...],
                      preferred_element_type=jnp.float32)
    ht = jnp.maximum(ht, 0.0)
    logit = jnp.dot(w2_ref[...], ht, preferred_element_type=jnp.float32)
    o_ref[...] = jax.nn.sigmoid(logit + b2_ref[...])


def _mlp_tc(ut, vt, w1a, w1b, w2, b2):
    return pl.pallas_call(
        _mlp_body,
        out_shape=jax.ShapeDtypeStruct((1, B), jnp.float32),
    )(ut, vt, w1a, w1b, w2, b2)


def kernel(x, W, H, W1, W2, b2):
    ui = x[:, 0].astype(jnp.int32)
    vi = x[:, 1].astype(jnp.int32)
    wt = jnp.transpose(W)
    ht = jnp.transpose(H)
    ut, vt = _make_gather_sc()(wt, ht, ui, vi)
    w1a = W1[:, :K]
    w1b = W1[:, K:]
    out = _mlp_tc(ut, vt, w1a, w1b, W2, b2.reshape(1, 1))
    return out.reshape(B)
